# baseline (speedup 1.0000x reference)
import functools

import jax
import jax.numpy as jnp
from jax import lax
from jax.experimental import pallas as pl
from jax.experimental.pallas import tpu as pltpu
from jax.experimental.pallas import tpu_sc as plsc

BATCH = 4096
HIST = 50
D = 128
NW = 32
BPW = BATCH // NW

_mesh = plsc.VectorSubcoreMesh(core_axis_name="c", subcore_axis_name="s")


NBUF = 4
MAIN = (HIST // NBUF) * NBUF


@functools.partial(
    pl.kernel,
    mesh=_mesh,
    out_type=jax.ShapeDtypeStruct((HIST, BATCH, D), jnp.float32),
    scratch_types=[
        pltpu.VMEM((HIST, BPW), jnp.int32),
    ]
    + [pltpu.VMEM((BPW, D), jnp.float32) for _ in range(NBUF)]
    + [pltpu.SemaphoreType.DMA for _ in range(2 * NBUF)],
)
def _gather_kernel(idx_hbm, table_hbm, out_hbm, idx_v, *bufs_and_sems):
    bufs = bufs_and_sems[:NBUF]
    gsem = bufs_and_sems[NBUF : 2 * NBUF]
    wsem = bufs_and_sems[2 * NBUF :]
    wid = lax.axis_index("s") * 2 + lax.axis_index("c")
    b0 = wid * BPW
    pltpu.sync_copy(idx_hbm.at[pl.ds(0, HIST), pl.ds(b0, BPW)], idx_v)

    def gather(h, k):
        return pltpu.make_async_copy(table_hbm.at[idx_v.at[h]], bufs[k], gsem[k])

    def writeback(h, k):
        return pltpu.make_async_copy(
            bufs[k], out_hbm.at[h, pl.ds(b0, BPW)], wsem[k]
        )

    for k in range(NBUF):
        gather(k, k).start()

    def body(i, carry):
        h0 = NBUF * i
        for k in range(NBUF):
            gather(h0 + k, k).wait()
            writeback(h0 + k, k).start()
        for k in range(NBUF):
            writeback(h0 + k, k).wait()

            @pl.when(h0 + NBUF + k < MAIN)
            def _():
                gather(h0 + NBUF + k, k).start()

        return carry

    lax.fori_loop(0, MAIN // NBUF, body, 0)

    for h in range(MAIN, HIST):
        k = h - MAIN
        gather(h, k).start()
        gather(h, k).wait()
        writeback(h, k).start()
        writeback(h, k).wait()


def kernel(input, weight):
    idx_t = jnp.transpose(input)
    out = _gather_kernel(idx_t, weight)
    return jnp.transpose(out, (1, 0, 2))


# speedup vs baseline: 1.0512x; 1.0512x over previous
import functools

import jax
import jax.numpy as jnp
from jax import lax
from jax.experimental import pallas as pl
from jax.experimental.pallas import tpu as pltpu
from jax.experimental.pallas import tpu_sc as plsc

BATCH = 4096
HIST = 50
D = 128
NW = 32
BPW = BATCH // NW

_mesh = plsc.VectorSubcoreMesh(core_axis_name="c", subcore_axis_name="s")


HG = 2
NCH = HIST // HG


@functools.partial(
    pl.kernel,
    mesh=_mesh,
    out_type=jax.ShapeDtypeStruct((HIST, BATCH, D), jnp.float32),
    scratch_types=[
        pltpu.VMEM((HIST, BPW), jnp.int32),
        pltpu.VMEM((HG, BPW, D), jnp.float32),
        pltpu.VMEM((HG, BPW, D), jnp.float32),
        pltpu.SemaphoreType.DMA,
        pltpu.SemaphoreType.DMA,
    ],
)
def _gather_kernel(idx_hbm, table_hbm, out_hbm, idx_v, buf0, buf1, g0, g1):
    wid = lax.axis_index("s") * 2 + lax.axis_index("c")
    b0 = wid * BPW
    pltpu.sync_copy(idx_hbm.at[pl.ds(0, HIST), pl.ds(b0, BPW)], idx_v)

    class gather:
        def __init__(self, c, buf, sem):
            self.copies = [
                pltpu.make_async_copy(
                    table_hbm.at[idx_v.at[c * HG + k]], buf.at[k], sem
                )
                for k in range(HG)
            ]

        def start(self):
            for cp in self.copies:
                cp.start()

        def wait(self):
            for cp in self.copies:
                cp.wait()

    def writeback(c, buf):
        pltpu.sync_copy(buf, out_hbm.at[pl.ds(c * HG, HG), pl.ds(b0, BPW)])

    gather(0, buf0, g0).start()

    def body(i, carry):
        c0 = 2 * i
        gather(c0 + 1, buf1, g1).start()
        gather(c0, buf0, g0).wait()
        writeback(c0, buf0)

        @pl.when(i < NCH // 2 - 1)
        def _():
            gather(c0 + 2, buf0, g0).start()

        gather(c0 + 1, buf1, g1).wait()
        writeback(c0 + 1, buf1)
        return carry

    lax.fori_loop(0, NCH // 2, body, 0)

    gather(NCH - 1, buf0, g0).start()
    gather(NCH - 1, buf0, g0).wait()
    writeback(NCH - 1, buf0)


def kernel(input, weight):
    idx_t = jnp.transpose(input)
    out = _gather_kernel(idx_t, weight)
    return jnp.transpose(out, (1, 0, 2))
